# baseline (speedup 1.0000x reference)
import functools

import jax
import jax.numpy as jnp
from jax import lax
from jax.experimental import pallas as pl
from jax.experimental.pallas import tpu as pltpu
from jax.experimental.pallas import tpu_sc as plsc

B = 4096
D = 128
C = 1000
M_ROWS = 100000

BI = 256
NBLK = B // BI

NC = 2
NS = 16
NW = NC * NS
CH = B // NW


NSTEP = 25
CP_ROWS = M_ROWS // NSTEP


def _mm(i):
  return jnp.minimum(i, NBLK - 1)


def _tc_body(idx_blk_ref, idx_all_ref, x_ref, w_ref, b_ref, mem_ref,
             out_ref, win_ref, newmem_ref):
  newmem_ref[...] = mem_ref[...]

  @pl.when(pl.program_id(0) < NBLK)
  def _compute():
    out_ref[...] = (
        jnp.dot(x_ref[...], w_ref[...], preferred_element_type=jnp.float32)
        + b_ref[...]
    )
    ii = idx_blk_ref[0, 0, :].reshape(BI, 1)
    alljj = idx_all_ref[...]
    eq = ii == alljj
    jio = lax.broadcasted_iota(jnp.int32, (BI, B), 1)
    win = jnp.max(jnp.where(eq, jio, -1), axis=1)
    win_ref[0, 0, :] = win


_tc_call = pl.pallas_call(
    _tc_body,
    grid=(NSTEP,),
    in_specs=[
        pl.BlockSpec((1, 1, BI), lambda i: (_mm(i), 0, 0)),
        pl.BlockSpec((1, B), lambda i: (0, 0)),
        pl.BlockSpec((BI, D), lambda i: (_mm(i), 0)),
        pl.BlockSpec((D, C), lambda i: (0, 0)),
        pl.BlockSpec((1, C), lambda i: (0, 0)),
        pl.BlockSpec((1, CP_ROWS, D), lambda i: (i, 0, 0)),
    ],
    out_specs=[
        pl.BlockSpec((BI, C), lambda i: (_mm(i), 0)),
        pl.BlockSpec((1, 1, BI), lambda i: (_mm(i), 0, 0)),
        pl.BlockSpec((1, CP_ROWS, D), lambda i: (i, 0, 0)),
    ],
    out_shape=[
        jax.ShapeDtypeStruct((B, C), jnp.float32),
        jax.ShapeDtypeStruct((NBLK, 1, BI), jnp.int32),
        jax.ShapeDtypeStruct((NSTEP, CP_ROWS, D), jnp.float32),
    ],
)


def _sc_scatter_body(x_hbm, idx_hbm, win_hbm, mem_hbm, idx_v, win_v, rows_v,
                     sem):
  wid = lax.axis_index("s") * NC + lax.axis_index("c")
  base = wid * CH
  pltpu.sync_copy(idx_hbm.at[pl.ds(base, CH)], idx_v)
  pltpu.sync_copy(win_hbm.at[pl.ds(base, CH)], win_v)
  pltpu.async_copy(x_hbm.at[win_v], rows_v, sem).wait()
  pltpu.async_copy(rows_v, mem_hbm.at[idx_v], sem).wait()


@functools.cache
def _sc_scatter():
  return functools.partial(
      pl.kernel,
      mesh=plsc.VectorSubcoreMesh(core_axis_name="c", subcore_axis_name="s"),
      scratch_types=[
          pltpu.VMEM((CH,), jnp.int32),
          pltpu.VMEM((CH,), jnp.int32),
          pltpu.VMEM((CH, D), jnp.float32),
          pltpu.SemaphoreType.DMA,
      ],
  )(_sc_scatter_body)


def kernel(x, mem, idx, W, b):
  idx32 = idx.astype(jnp.int32)
  out, win3, new_mem3 = _tc_call(
      idx32.reshape(NBLK, 1, BI),
      idx32.reshape(1, B),
      x,
      W,
      b.reshape(1, C),
      mem.reshape(NSTEP, CP_ROWS, D),
  )
  winner = win3.reshape(B)
  mem_ref = jax.new_ref(new_mem3.reshape(M_ROWS, D))
  _sc_scatter()(x, idx32, winner, mem_ref)
  return out, mem_ref[...]


# speedup vs baseline: 1.0787x; 1.0787x over previous
import functools

import jax
import jax.numpy as jnp
from jax import lax
from jax.experimental import pallas as pl
from jax.experimental.pallas import tpu as pltpu
from jax.experimental.pallas import tpu_sc as plsc

B = 4096
D = 128
C = 1000
M_ROWS = 100000

BI = 512
NBLK = B // BI

NC = 2
NS = 16
NW = NC * NS
CH = B // NW


NSTEP = 10
CP_ROWS = M_ROWS // NSTEP


def _mm(i):
  return jnp.minimum(i, NBLK - 1)


def _tc_body(idx_blk_ref, idx_all_ref, x_ref, w_ref, b_ref, mem_ref,
             out_ref, win_ref, newmem_ref):
  newmem_ref[...] = mem_ref[...]

  @pl.when(pl.program_id(0) < NBLK)
  def _compute():
    out_ref[...] = (
        jnp.dot(x_ref[...], w_ref[...], preferred_element_type=jnp.float32)
        + b_ref[...]
    )
    ii = idx_blk_ref[0, 0, :].reshape(BI, 1)
    alljj = idx_all_ref[...]
    eq = ii == alljj
    jio = lax.broadcasted_iota(jnp.int32, (BI, B), 1)
    win = jnp.max(jnp.where(eq, jio, -1), axis=1)
    win_ref[0, 0, :] = win


_tc_call = pl.pallas_call(
    _tc_body,
    grid=(NSTEP,),
    in_specs=[
        pl.BlockSpec((1, 1, BI), lambda i: (_mm(i), 0, 0)),
        pl.BlockSpec((1, B), lambda i: (0, 0)),
        pl.BlockSpec((BI, D), lambda i: (_mm(i), 0)),
        pl.BlockSpec((D, C), lambda i: (0, 0)),
        pl.BlockSpec((1, C), lambda i: (0, 0)),
        pl.BlockSpec((1, CP_ROWS, D), lambda i: (i, 0, 0)),
    ],
    out_specs=[
        pl.BlockSpec((BI, C), lambda i: (_mm(i), 0)),
        pl.BlockSpec((1, 1, BI), lambda i: (_mm(i), 0, 0)),
        pl.BlockSpec((1, CP_ROWS, D), lambda i: (i, 0, 0)),
    ],
    out_shape=[
        jax.ShapeDtypeStruct((B, C), jnp.float32),
        jax.ShapeDtypeStruct((NBLK, 1, BI), jnp.int32),
        jax.ShapeDtypeStruct((NSTEP, CP_ROWS, D), jnp.float32),
    ],
)


def _sc_scatter_body(x_hbm, idx_hbm, win_hbm, mem_hbm, idx_v, win_v, rows_v,
                     sem):
  wid = lax.axis_index("s") * NC + lax.axis_index("c")
  base = wid * CH
  pltpu.sync_copy(idx_hbm.at[pl.ds(base, CH)], idx_v)
  pltpu.sync_copy(win_hbm.at[pl.ds(base, CH)], win_v)
  pltpu.async_copy(x_hbm.at[win_v], rows_v, sem).wait()
  pltpu.async_copy(rows_v, mem_hbm.at[idx_v], sem).wait()


@functools.cache
def _sc_scatter():
  return functools.partial(
      pl.kernel,
      mesh=plsc.VectorSubcoreMesh(core_axis_name="c", subcore_axis_name="s"),
      scratch_types=[
          pltpu.VMEM((CH,), jnp.int32),
          pltpu.VMEM((CH,), jnp.int32),
          pltpu.VMEM((CH, D), jnp.float32),
          pltpu.SemaphoreType.DMA,
      ],
  )(_sc_scatter_body)


def kernel(x, mem, idx, W, b):
  idx32 = idx.astype(jnp.int32)
  out, win3, new_mem3 = _tc_call(
      idx32.reshape(NBLK, 1, BI),
      idx32.reshape(1, B),
      x,
      W,
      b.reshape(1, C),
      mem.reshape(NSTEP, CP_ROWS, D),
  )
  winner = win3.reshape(B)
  mem_ref = jax.new_ref(new_mem3.reshape(M_ROWS, D))
  _sc_scatter()(x, idx32, winner, mem_ref)
  return out, mem_ref[...]
